# Initial kernel scaffold; baseline (speedup 1.0000x reference)
import jax
import jax.numpy as jnp
from jax.experimental import pallas as pl


def kernel(indices, emb, W, b):
    raise NotImplementedError("write your pallas kernel here")


# speedup vs baseline: 94.8966x; 94.8966x over previous
import functools

import jax
import jax.numpy as jnp
from jax import lax
from jax.experimental import pallas as pl
from jax.experimental.pallas import tpu as pltpu
from jax.experimental.pallas import tpu_sc as plsc

_VOCAB = 1000
_VOCAB_PAD = 1024
_PAD = 1


def _table_body(emb_ref, w_ref, b_ref, out_ref):
    t = jnp.dot(emb_ref[...], w_ref[...], preferred_element_type=jnp.float32)
    row = lax.broadcasted_iota(jnp.int32, t.shape, 0)
    out_ref[...] = jnp.where(row == _PAD, 0.0, t) + b_ref[...]


def _build_table(emb_padded, w, b2):
    return pl.pallas_call(
        _table_body,
        out_shape=jax.ShapeDtypeStruct((_VOCAB_PAD, 1), jnp.float32),
    )(emb_padded, w, b2)


def _sc_lookup(table, idx):
    info = plsc.get_sparse_core_info()
    nw = info.num_cores * info.num_subcores
    lanes = info.num_lanes
    batch = idx.shape[0]
    bpw = batch // nw
    mesh = plsc.VectorSubcoreMesh(core_axis_name="c", subcore_axis_name="s")

    @functools.partial(
        pl.kernel,
        out_type=jax.ShapeDtypeStruct((batch,), jnp.float32),
        mesh=mesh,
        scratch_types=[
            pltpu.VMEM((_VOCAB_PAD,), jnp.float32),
            pltpu.VMEM((bpw,), jnp.int32),
            pltpu.VMEM((bpw,), jnp.float32),
        ],
        compiler_params=pltpu.CompilerParams(needs_layout_passes=False),
    )
    def k(table_hbm, idx_hbm, out_hbm, table_v, idx_v, out_v):
        wid = lax.axis_index("s") * info.num_cores + lax.axis_index("c")
        base = wid * bpw
        pltpu.sync_copy(table_hbm, table_v)
        pltpu.sync_copy(idx_hbm.at[pl.ds(base, bpw)], idx_v)
        for j in range(bpw // lanes):
            iv = idx_v[pl.ds(j * lanes, lanes)]
            out_v[pl.ds(j * lanes, lanes)] = plsc.load_gather(table_v, [iv])
        pltpu.sync_copy(out_v, out_hbm.at[pl.ds(base, bpw)])

    return k(table, idx)


def kernel(indices, emb, W, b):
    idx0 = indices[0, :, 0].astype(jnp.int32)
    emb_p = jnp.pad(emb, ((0, _VOCAB_PAD - _VOCAB), (0, 0)))
    table = _build_table(emb_p, W, b.reshape(1, 1))
    out = _sc_lookup(table[:, 0], idx0)
    return out[:, None]
